# Initial kernel scaffold; baseline (speedup 1.0000x reference)
import jax
import jax.numpy as jnp
from jax.experimental import pallas as pl


def kernel(latents, prototypes):
    raise NotImplementedError("write your pallas kernel here")


# speedup vs baseline: 1.1792x; 1.1792x over previous
import functools

import jax
import jax.numpy as jnp
from jax import lax
from jax.experimental import pallas as pl
from jax.experimental.pallas import tpu as pltpu
from jax.experimental.pallas import tpu_sc as plsc

N = 16384
K = 1024
D = 64
BETA = 0.25
BLK = 1024
NB = N // BLK

_NC = 2
_NS = 16
_NW = _NC * _NS
_ROWS_PER_W = N // _NW
_CHUNK = 128
_NCHUNK = _ROWS_PER_W // _CHUNK


def _vq_body(x_ref, pt_ref, x2_ref, p2_ref, idx_ref, vq_ref, ent_ref,
             acc_ref, vqacc_ref):
    i = pl.program_id(0)
    x = x_ref[...]
    pt = pt_ref[...]
    dots = jnp.dot(x, pt, preferred_element_type=jnp.float32)
    dists = (x2_ref[...] + p2_ref[...]) - 2.0 * dots
    mind = jnp.min(dists, axis=1, keepdims=True)
    iota = lax.broadcasted_iota(jnp.int32, (BLK, K), 1)
    idx_ref[...] = jnp.min(
        jnp.where(dists == mind, iota, K), axis=1, keepdims=True)

    e = jnp.exp(mind - dists)
    z = jnp.sum(e, axis=1, keepdims=True)
    probs = e * (1.0 / z)

    @pl.when(i == 0)
    def _init():
        acc_ref[...] = jnp.zeros_like(acc_ref)
        vqacc_ref[0, 0] = 0.0

    acc_ref[...] += jnp.sum(probs, axis=0, keepdims=True)
    vqacc_ref[0, 0] += jnp.sum(mind)

    @pl.when(i == NB - 1)
    def _fin():
        s = acc_ref[...] * (1.0 / N) + 1e-8
        s = s / jnp.sum(s)
        ent_ref[...] = jnp.sum(-s * jnp.log(s), keepdims=True).reshape(1, 1)
        vq_ref[...] = jnp.full(
            (1, 1), (1.0 + BETA) * vqacc_ref[0, 0] / (N * D), jnp.float32)


_vq_call = pl.pallas_call(
    _vq_body,
    grid=(NB,),
    in_specs=[
        pl.BlockSpec((BLK, D), lambda i: (i, 0)),
        pl.BlockSpec((D, K), lambda i: (0, 0)),
        pl.BlockSpec((BLK, 1), lambda i: (i, 0)),
        pl.BlockSpec((1, K), lambda i: (0, 0)),
    ],
    out_specs=[
        pl.BlockSpec((BLK, 1), lambda i: (i, 0)),
        pl.BlockSpec((1, 1), lambda i: (0, 0)),
        pl.BlockSpec((1, 1), lambda i: (0, 0)),
    ],
    out_shape=[
        jax.ShapeDtypeStruct((N, 1), jnp.int32),
        jax.ShapeDtypeStruct((1, 1), jnp.float32),
        jax.ShapeDtypeStruct((1, 1), jnp.float32),
    ],
    scratch_shapes=[
        pltpu.VMEM((1, K), jnp.float32),
        pltpu.SMEM((1, 1), jnp.float32),
    ],
)


@functools.cache
def _sc_gather_call():
    @functools.partial(
        pl.kernel,
        mesh=plsc.VectorSubcoreMesh(core_axis_name="c", subcore_axis_name="s"),
        out_type=jax.ShapeDtypeStruct((N, D), jnp.float32),
        scratch_types=[
            pltpu.VMEM((_NCHUNK, _CHUNK), jnp.int32),
            pltpu.VMEM((_ROWS_PER_W, D), jnp.float32),
            pltpu.SemaphoreType.DMA,
        ],
        compiler_params=pltpu.CompilerParams(use_tc_tiling_on_sc=False),
    )
    def _sc_gather(table_hbm, idx_hbm, out_hbm, idx_v, rows_v, sem):
        wid = lax.axis_index("s") * _NC + lax.axis_index("c")
        pltpu.sync_copy(idx_hbm.at[pl.ds(wid * _NCHUNK, _NCHUNK)], idx_v)
        copies = []
        for j in range(_NCHUNK):
            copies.append(pltpu.async_copy(
                table_hbm.at[idx_v.at[j]],
                rows_v.at[pl.ds(j * _CHUNK, _CHUNK)], sem))
        for c in copies:
            c.wait()
        pltpu.sync_copy(
            rows_v, out_hbm.at[pl.ds(wid * _ROWS_PER_W, _ROWS_PER_W)])

    return _sc_gather


def kernel(latents, prototypes):
    x2 = jnp.sum(latents ** 2, axis=1, keepdims=True)
    p2 = jnp.sum(prototypes ** 2, axis=1).reshape(1, K)
    idx, vq, ent = _vq_call(latents, prototypes.T, x2, p2)
    quantized = _sc_gather_call()(
        prototypes, idx.reshape(_NW * _NCHUNK, _CHUNK))
    return quantized, vq[0, 0], ent[0, 0]


# speedup vs baseline: 1.1841x; 1.0042x over previous
import functools

import jax
import jax.numpy as jnp
from jax import lax
from jax.experimental import pallas as pl
from jax.experimental.pallas import tpu as pltpu
from jax.experimental.pallas import tpu_sc as plsc

N = 16384
K = 1024
D = 64
BETA = 0.25
BLK = 2048
NB = N // BLK

_NC = 2
_NS = 16
_NW = _NC * _NS
_ROWS_PER_W = N // _NW
_CHUNK = 128
_NCHUNK = _ROWS_PER_W // _CHUNK


def _vq_body(x_ref, pt_ref, x2_ref, p2_ref, iota_ref, idx_ref, vq_ref,
             ent_ref, acc_ref, vqacc_ref):
    i = pl.program_id(0)
    x = x_ref[...]
    pt = pt_ref[...]
    dots = jnp.dot(x, pt, preferred_element_type=jnp.float32)
    dists = (x2_ref[...] + p2_ref[...]) + dots
    mind = jnp.min(dists, axis=1, keepdims=True)
    idx_f = jnp.min(jnp.where(dists == mind, iota_ref[...], float(K)),
                    axis=1, keepdims=True)
    idx_ref[...] = idx_f.astype(jnp.int32)

    e = jnp.exp(mind - dists)
    z = jnp.sum(e, axis=1, keepdims=True)

    @pl.when(i == 0)
    def _init():
        acc_ref[...] = jnp.zeros_like(acc_ref)
        vqacc_ref[0, 0] = 0.0

    acc_ref[...] += jnp.sum(e * (1.0 / z), axis=0, keepdims=True)
    vqacc_ref[0, 0] += jnp.sum(mind)

    @pl.when(i == NB - 1)
    def _fin():
        s = acc_ref[...] * (1.0 / N) + 1e-8
        s = s / jnp.sum(s)
        ent_ref[...] = jnp.sum(-s * jnp.log(s), keepdims=True).reshape(1, 1)
        vq_ref[...] = jnp.full(
            (1, 1), (1.0 + BETA) * vqacc_ref[0, 0] / (N * D), jnp.float32)


_vq_call = pl.pallas_call(
    _vq_body,
    grid=(NB,),
    in_specs=[
        pl.BlockSpec((BLK, D), lambda i: (i, 0)),
        pl.BlockSpec((D, K), lambda i: (0, 0)),
        pl.BlockSpec((BLK, 1), lambda i: (i, 0)),
        pl.BlockSpec((1, K), lambda i: (0, 0)),
        pl.BlockSpec((1, K), lambda i: (0, 0)),
    ],
    out_specs=[
        pl.BlockSpec((BLK, 1), lambda i: (i, 0)),
        pl.BlockSpec((1, 1), lambda i: (0, 0)),
        pl.BlockSpec((1, 1), lambda i: (0, 0)),
    ],
    out_shape=[
        jax.ShapeDtypeStruct((N, 1), jnp.int32),
        jax.ShapeDtypeStruct((1, 1), jnp.float32),
        jax.ShapeDtypeStruct((1, 1), jnp.float32),
    ],
    scratch_shapes=[
        pltpu.VMEM((1, K), jnp.float32),
        pltpu.SMEM((1, 1), jnp.float32),
    ],
)


@functools.cache
def _sc_gather_call():
    @functools.partial(
        pl.kernel,
        mesh=plsc.VectorSubcoreMesh(core_axis_name="c", subcore_axis_name="s"),
        out_type=jax.ShapeDtypeStruct((N, D), jnp.float32),
        scratch_types=[
            pltpu.VMEM((_NCHUNK, _CHUNK), jnp.int32),
            pltpu.VMEM((_ROWS_PER_W, D), jnp.float32),
            pltpu.SemaphoreType.DMA,
        ],
        compiler_params=pltpu.CompilerParams(use_tc_tiling_on_sc=False),
    )
    def _sc_gather(table_hbm, idx_hbm, out_hbm, idx_v, rows_v, sem):
        wid = lax.axis_index("s") * _NC + lax.axis_index("c")
        pltpu.sync_copy(idx_hbm.at[pl.ds(wid * _NCHUNK, _NCHUNK)], idx_v)
        copies = []
        for j in range(_NCHUNK):
            copies.append(pltpu.async_copy(
                table_hbm.at[idx_v.at[j]],
                rows_v.at[pl.ds(j * _CHUNK, _CHUNK)], sem))
        for c in copies:
            c.wait()
        pltpu.sync_copy(
            rows_v, out_hbm.at[pl.ds(wid * _ROWS_PER_W, _ROWS_PER_W)])

    return _sc_gather


def kernel(latents, prototypes):
    x2 = jnp.sum(latents ** 2, axis=1, keepdims=True)
    p2 = jnp.sum(prototypes ** 2, axis=1).reshape(1, K)
    iota_row = lax.broadcasted_iota(jnp.float32, (1, K), 1)
    idx, vq, ent = _vq_call(latents, -2.0 * prototypes.T, x2, p2, iota_row)
    quantized = _sc_gather_call()(
        prototypes, idx.reshape(_NW * _NCHUNK, _CHUNK))
    return quantized, vq[0, 0], ent[0, 0]


# speedup vs baseline: 1.2070x; 1.0193x over previous
import functools

import jax
import jax.numpy as jnp
from jax import lax
from jax.experimental import pallas as pl
from jax.experimental.pallas import tpu as pltpu
from jax.experimental.pallas import tpu_sc as plsc

N = 16384
K = 1024
D = 64
BETA = 0.25
BLK = 2048
NB = N // BLK

_NC = 2
_NS = 16
_NW = _NC * _NS
_ROWS_PER_W = N // _NW
_CHUNK = 128
_NCHUNK = _ROWS_PER_W // _CHUNK


def _vq_body(x_ref, pt_ref, x2_ref, p2_ref, iota_ref, idx_ref, vq_ref,
             ent_ref, acc_ref, vqacc_ref):
    i = pl.program_id(0)
    x = x_ref[...]
    pt = pt_ref[...]
    dots = jnp.dot(x, pt, preferred_element_type=jnp.float32)
    dists = (x2_ref[...] + p2_ref[...]) + dots
    mind = jnp.min(dists, axis=1, keepdims=True)
    idx_f = jnp.min(jnp.where(dists == mind, iota_ref[...], float(K)),
                    axis=1, keepdims=True)
    idx_ref[...] = idx_f.astype(jnp.int32)

    e = jnp.exp(mind - dists)
    z = jnp.sum(e, axis=1, keepdims=True)

    @pl.when(i == 0)
    def _init():
        acc_ref[...] = jnp.zeros_like(acc_ref)
        vqacc_ref[0, 0] = 0.0

    acc_ref[...] += jnp.sum(e * (1.0 / z), axis=0, keepdims=True)
    vqacc_ref[0, 0] += jnp.sum(mind)

    @pl.when(i == NB - 1)
    def _fin():
        s = acc_ref[...] * (1.0 / N) + 1e-8
        s = s / jnp.sum(s)
        ent_ref[...] = jnp.sum(-s * jnp.log(s), keepdims=True).reshape(1, 1)
        vq_ref[...] = jnp.full(
            (1, 1), (1.0 + BETA) * vqacc_ref[0, 0] / (N * D), jnp.float32)


_vq_call = pl.pallas_call(
    _vq_body,
    grid=(NB,),
    in_specs=[
        pl.BlockSpec((BLK, D), lambda i: (i, 0)),
        pl.BlockSpec((D, K), lambda i: (0, 0)),
        pl.BlockSpec((BLK, 1), lambda i: (i, 0)),
        pl.BlockSpec((1, K), lambda i: (0, 0)),
        pl.BlockSpec((1, K), lambda i: (0, 0)),
    ],
    out_specs=[
        pl.BlockSpec((BLK, 1), lambda i: (i, 0)),
        pl.BlockSpec((1, 1), lambda i: (0, 0)),
        pl.BlockSpec((1, 1), lambda i: (0, 0)),
    ],
    out_shape=[
        jax.ShapeDtypeStruct((N, 1), jnp.int32),
        jax.ShapeDtypeStruct((1, 1), jnp.float32),
        jax.ShapeDtypeStruct((1, 1), jnp.float32),
    ],
    scratch_shapes=[
        pltpu.VMEM((1, K), jnp.float32),
        pltpu.SMEM((1, 1), jnp.float32),
    ],
)


@functools.cache
def _sc_gather_call():
    @functools.partial(
        pl.kernel,
        mesh=plsc.VectorSubcoreMesh(core_axis_name="c", subcore_axis_name="s"),
        out_type=jax.ShapeDtypeStruct((N, 2 * D), jnp.float32),
        scratch_types=[
            pltpu.VMEM((_NCHUNK, _CHUNK), jnp.int32),
            pltpu.VMEM((_ROWS_PER_W, 2 * D), jnp.float32),
            pltpu.SemaphoreType.DMA,
        ],
    )
    def _sc_gather(table_hbm, idx_hbm, out_hbm, idx_v, rows_v, sem):
        wid = lax.axis_index("s") * _NC + lax.axis_index("c")
        pltpu.sync_copy(idx_hbm.at[pl.ds(wid * _NCHUNK, _NCHUNK)], idx_v)
        copies = []
        for j in range(_NCHUNK):
            copies.append(pltpu.async_copy(
                table_hbm.at[idx_v.at[j]],
                rows_v.at[pl.ds(j * _CHUNK, _CHUNK)], sem))
        for c in copies:
            c.wait()
        pltpu.sync_copy(
            rows_v, out_hbm.at[pl.ds(wid * _ROWS_PER_W, _ROWS_PER_W)])

    return _sc_gather


def kernel(latents, prototypes):
    x2 = jnp.sum(latents ** 2, axis=1, keepdims=True)
    p2 = jnp.sum(prototypes ** 2, axis=1).reshape(1, K)
    iota_row = lax.broadcasted_iota(jnp.float32, (1, K), 1)
    idx, vq, ent = _vq_call(latents, -2.0 * prototypes.T, x2, p2, iota_row)
    table = jnp.pad(prototypes, ((0, 0), (0, D)))
    gathered = _sc_gather_call()(
        table, idx.reshape(_NW * _NCHUNK, _CHUNK))
    return gathered[:, :D], vq[0, 0], ent[0, 0]


# speedup vs baseline: 1.4304x; 1.1851x over previous
import functools

import jax
import jax.numpy as jnp
from jax import lax
from jax.experimental import pallas as pl
from jax.experimental.pallas import tpu as pltpu
from jax.experimental.pallas import tpu_sc as plsc

N = 16384
K = 1024
D = 64
BETA = 0.25
BLK = 2048
NB = N // BLK

_NC = 2
_NS = 16
_NW = _NC * _NS
_ROWS_PER_W = N // _NW
_CHUNK = 128
_NCHUNK = _ROWS_PER_W // _CHUNK


def _vq_body(xt_ref, pt_ref, x2_ref, p2_ref, iota_ref, idx_ref, vq_ref,
             ent_ref, acc_ref, vqacc_ref):
    i = pl.program_id(0)
    xt = xt_ref[...]
    pt = pt_ref[...]
    dots = lax.dot_general(
        xt, pt, (((0,), (0,)), ((), ())),
        preferred_element_type=jnp.float32)
    x2 = x2_ref[...].reshape(BLK, 1)
    dists = (x2 + p2_ref[...]) + dots
    mind = jnp.min(dists, axis=1, keepdims=True)
    idx_f = jnp.min(jnp.where(dists == mind, iota_ref[...], float(K)),
                    axis=1, keepdims=True)
    idx_ref[...] = idx_f.astype(jnp.int32).reshape(BLK // 128, 128)

    e = jnp.exp(mind - dists)
    z = jnp.sum(e, axis=1, keepdims=True)

    @pl.when(i == 0)
    def _init():
        acc_ref[...] = jnp.zeros_like(acc_ref)
        vqacc_ref[0, 0] = 0.0

    acc_ref[...] += jnp.sum(e * (1.0 / z), axis=0, keepdims=True)
    vqacc_ref[0, 0] += jnp.sum(mind)

    @pl.when(i == NB - 1)
    def _fin():
        s = acc_ref[...] * (1.0 / N) + 1e-8
        s = s / jnp.sum(s)
        ent_ref[...] = jnp.sum(-s * jnp.log(s), keepdims=True).reshape(1, 1)
        vq_ref[...] = jnp.full(
            (1, 1), (1.0 + BETA) * vqacc_ref[0, 0] / (N * D), jnp.float32)


_vq_call = pl.pallas_call(
    _vq_body,
    grid=(NB,),
    in_specs=[
        pl.BlockSpec((D, BLK), lambda i: (0, i)),
        pl.BlockSpec((D, K), lambda i: (0, 0)),
        pl.BlockSpec((BLK,), lambda i: (i,)),
        pl.BlockSpec((1, K), lambda i: (0, 0)),
        pl.BlockSpec((1, K), lambda i: (0, 0)),
    ],
    out_specs=[
        pl.BlockSpec((BLK // 128, 128), lambda i: (i, 0)),
        pl.BlockSpec((1, 1), lambda i: (0, 0)),
        pl.BlockSpec((1, 1), lambda i: (0, 0)),
    ],
    out_shape=[
        jax.ShapeDtypeStruct((N // 128, 128), jnp.int32),
        jax.ShapeDtypeStruct((1, 1), jnp.float32),
        jax.ShapeDtypeStruct((1, 1), jnp.float32),
    ],
    scratch_shapes=[
        pltpu.VMEM((1, K), jnp.float32),
        pltpu.SMEM((1, 1), jnp.float32),
    ],
)


@functools.cache
def _sc_gather_call():
    @functools.partial(
        pl.kernel,
        mesh=plsc.VectorSubcoreMesh(core_axis_name="c", subcore_axis_name="s"),
        out_type=jax.ShapeDtypeStruct((N, 2 * D), jnp.float32),
        scratch_types=[
            pltpu.VMEM((_NCHUNK, _CHUNK), jnp.int32),
            pltpu.VMEM((_ROWS_PER_W, 2 * D), jnp.float32),
            pltpu.SemaphoreType.DMA,
        ],
    )
    def _sc_gather(table_hbm, idx_hbm, out_hbm, idx_v, rows_v, sem):
        wid = lax.axis_index("s") * _NC + lax.axis_index("c")
        pltpu.sync_copy(idx_hbm.at[pl.ds(wid * _NCHUNK, _NCHUNK)], idx_v)
        copies = []
        for j in range(_NCHUNK):
            copies.append(pltpu.async_copy(
                table_hbm.at[idx_v.at[j]],
                rows_v.at[pl.ds(j * _CHUNK, _CHUNK)], sem))
        for c in copies:
            c.wait()
        pltpu.sync_copy(
            rows_v, out_hbm.at[pl.ds(wid * _ROWS_PER_W, _ROWS_PER_W)])

    return _sc_gather


def kernel(latents, prototypes):
    x2 = jnp.sum(latents ** 2, axis=1)
    p2 = jnp.sum(prototypes ** 2, axis=1).reshape(1, K)
    iota_row = lax.broadcasted_iota(jnp.float32, (1, K), 1)
    idx, vq, ent = _vq_call(latents.T, -2.0 * prototypes.T, x2, p2, iota_row)
    table = jnp.pad(prototypes, ((0, 0), (0, D)))
    gathered = _sc_gather_call()(table, idx)
    return gathered[:, :D], vq[0, 0], ent[0, 0]
